# Initial kernel scaffold; baseline (speedup 1.0000x reference)
import jax
import jax.numpy as jnp
from jax.experimental import pallas as pl


def kernel(atomic_numbers, edge_index, lengths, atomic_energies, c_zbl_raw, d_zbl_raw, zbl_pow_raw, zbl_length_raw):
    raise NotImplementedError("write your pallas kernel here")


# speedup vs baseline: 293.2566x; 293.2566x over previous
import jax
import jax.numpy as jnp
import numpy as np
from jax import lax
from jax.experimental import pallas as pl
from jax.experimental.pallas import tpu as pltpu
from jax.experimental.pallas import tpu_sc as plsc

N_NODES = 100000
N_EDGES = 6400000
KE = 14.399645351950548
_COV = np.linspace(0.2, 2.6, 119).astype(np.float32)

NC, NS, L = 2, 16, 16
NW = NC * NS
ROW_W = 128
ROWS = N_EDGES // ROW_W
CHUNK_ROWS = 16
ROWS_PER_W = 1568
NCH_FULL = ROWS_PER_W // CHUNK_ROWS
NCH_LAST = (ROWS - (NW - 1) * ROWS_PER_W) // CHUNK_ROWS
ACC_PAD = 100352
SLICE = ACC_PAD // NS
P2W = ACC_PAD // NW


def _edge_body(an_h, ii_h, ij_h, len_h, pt_h, ct_h, par_h, out_h,
               an_v, pt_v, ct_v, par_v, ii_v, ij_v, len_v, val_v,
               stage_v, acc_sh):
    cid = lax.axis_index("c")
    sid = lax.axis_index("s")
    w = cid * NS + sid

    pltpu.sync_copy(an_h, an_v)
    pltpu.sync_copy(pt_h, pt_v)
    pltpu.sync_copy(ct_h, ct_v)
    pltpu.sync_copy(par_h, par_v)

    zero = jnp.zeros((L,), jnp.float32)

    def zbody(i, carry):
        stage_v[pl.ds(i * L, L)] = zero
        return carry

    lax.fori_loop(0, SLICE // L, zbody, 0)
    pltpu.sync_copy(stage_v, acc_sh.at[pl.ds(sid * SLICE, SLICE)])
    plsc.subcore_barrier()

    c0 = par_v[0]
    c1 = par_v[1]
    c2 = par_v[2]
    c3 = par_v[3]
    nd0 = par_v[4]
    nd1 = par_v[5]
    nd2 = par_v[6]
    nd3 = par_v[7]
    ke2 = par_v[8]
    inv_zl = par_v[9]
    one = jnp.ones((L,), jnp.float32)
    zerov = jnp.zeros((L,), jnp.float32)

    base_row = w * ROWS_PER_W
    n_ch = jnp.where(w == NW - 1, NCH_LAST, NCH_FULL)

    def chunk_body(g, carry):
        row0 = base_row + g * CHUNK_ROWS
        pltpu.sync_copy(ii_h.at[pl.ds(row0, CHUNK_ROWS)], ii_v)
        pltpu.sync_copy(ij_h.at[pl.ds(row0, CHUNK_ROWS)], ij_v)
        pltpu.sync_copy(len_h.at[pl.ds(row0, CHUNK_ROWS)], len_v)
        for r in range(CHUNK_ROWS):
            def vbody(v, c2_):
                sl = pl.ds(v * L, L)
                ii = ii_v[r, sl]
                ij = ij_v[r, sl]
                ln = len_v[r, sl]
                zi = plsc.load_gather(an_v, [ii])
                zj = plsc.load_gather(an_v, [ij])
                pi = plsc.load_gather(pt_v, [zi])
                pj = plsc.load_gather(pt_v, [zj])
                ci = plsc.load_gather(ct_v, [zi])
                cj = plsc.load_gather(ct_v, [zj])
                zif = zi.astype(jnp.float32)
                zjf = zj.astype(jnp.float32)
                t = ln * (pi + pj) * inv_zl
                f = (c0 * jnp.exp(nd0 * t) + c1 * jnp.exp(nd1 * t)
                     + c2 * jnp.exp(nd2 * t) + c3 * jnp.exp(nd3 * t))
                pref = ke2 * zif * zjf / ln
                x = ln / (ci + cj)
                x2 = x * x
                x6 = x2 * x2 * x2
                poly = one + x6 * (x * 48.0 - x2 * 21.0 - 28.0)
                val = jnp.where(x < one, pref * f * poly, zerov)
                val_v[r, sl] = val
                return c2_

            lax.fori_loop(0, ROW_W // L, vbody, 0)
        for r in range(CHUNK_ROWS):
            pltpu.sync_copy(val_v.at[r], acc_sh.at[ii_v.at[r]], add=True)
        return carry

    lax.fori_loop(0, n_ch, chunk_body, 0)

    plsc.subcore_barrier()
    pltpu.sync_copy(acc_sh.at[pl.ds(sid * SLICE, SLICE)], stage_v)
    pltpu.sync_copy(stage_v, out_h.at[pl.ds(cid * ACC_PAD + sid * SLICE, SLICE)])


def _reduce_body(p_h, ae_h, zbl_h, tot_h, a_v, b_v, e_v):
    cid = lax.axis_index("c")
    sid = lax.axis_index("s")
    w = cid * NS + sid
    off = w * P2W
    pltpu.sync_copy(p_h.at[pl.ds(off, P2W)], a_v)
    pltpu.sync_copy(p_h.at[pl.ds(ACC_PAD + off, P2W)], b_v)
    pltpu.sync_copy(ae_h.at[pl.ds(off, P2W)], e_v)

    def vb(v, carry):
        sl = pl.ds(v * L, L)
        z = a_v[sl] + b_v[sl]
        a_v[sl] = z
        e_v[sl] = e_v[sl] + z
        return carry

    lax.fori_loop(0, P2W // L, vb, 0)
    pltpu.sync_copy(a_v, zbl_h.at[pl.ds(off, P2W)])
    pltpu.sync_copy(e_v, tot_h.at[pl.ds(off, P2W)])


_MESH = plsc.VectorSubcoreMesh(core_axis_name="c", subcore_axis_name="s",
                               num_cores=NC, num_subcores=NS)
_CPARAMS = pltpu.CompilerParams(needs_layout_passes=False)

_phase1 = pl.kernel(
    _edge_body,
    out_type=jax.ShapeDtypeStruct((NC * ACC_PAD,), jnp.float32),
    mesh=_MESH,
    scratch_types=[
        pltpu.VMEM((ACC_PAD,), jnp.int32),
        pltpu.VMEM((128,), jnp.float32),
        pltpu.VMEM((128,), jnp.float32),
        pltpu.VMEM((16, L), jnp.float32),
        pltpu.VMEM((CHUNK_ROWS, ROW_W), jnp.int32),
        pltpu.VMEM((CHUNK_ROWS, ROW_W), jnp.int32),
        pltpu.VMEM((CHUNK_ROWS, ROW_W), jnp.float32),
        pltpu.VMEM((CHUNK_ROWS, ROW_W), jnp.float32),
        pltpu.VMEM((SLICE,), jnp.float32),
        pltpu.VMEM_SHARED((ACC_PAD,), jnp.float32),
    ],
    compiler_params=_CPARAMS,
)

_phase2 = pl.kernel(
    _reduce_body,
    out_type=(jax.ShapeDtypeStruct((ACC_PAD,), jnp.float32),
              jax.ShapeDtypeStruct((ACC_PAD,), jnp.float32)),
    mesh=_MESH,
    scratch_types=[
        pltpu.VMEM((P2W,), jnp.float32),
        pltpu.VMEM((P2W,), jnp.float32),
        pltpu.VMEM((P2W,), jnp.float32),
    ],
    compiler_params=_CPARAMS,
)


def kernel(atomic_numbers, edge_index, lengths, atomic_energies,
           c_zbl_raw, d_zbl_raw, zbl_pow_raw, zbl_length_raw):
    an = jnp.zeros((ACC_PAD,), jnp.int32).at[:N_NODES].set(
        atomic_numbers.astype(jnp.int32))
    ii = edge_index[0].astype(jnp.int32).reshape(ROWS, ROW_W)
    ij = edge_index[1].astype(jnp.int32).reshape(ROWS, ROW_W)
    ln = lengths.reshape(ROWS, ROW_W).astype(jnp.float32)

    c = jax.nn.softplus(c_zbl_raw)
    c = c / c.sum()
    d = jax.nn.softplus(d_zbl_raw)
    p = jax.nn.softplus(zbl_pow_raw)[0]
    zl = jax.nn.softplus(zbl_length_raw)[0]

    powtab = jnp.arange(128, dtype=jnp.float32) ** p
    covtab = jnp.zeros((128,), jnp.float32).at[:119].set(jnp.asarray(_COV))
    scal = jnp.stack([c[0], c[1], c[2], c[3], -d[0], -d[1], -d[2], -d[3],
                      jnp.float32(KE / 2.0), 1.0 / zl,
                      jnp.float32(0.0), jnp.float32(0.0), jnp.float32(0.0),
                      jnp.float32(0.0), jnp.float32(0.0), jnp.float32(0.0)])
    par = jnp.repeat(scal[:, None], L, axis=1)

    ae_pad = jnp.zeros((ACC_PAD,), jnp.float32).at[:N_NODES].set(atomic_energies)

    partial = _phase1(an, ii, ij, ln, powtab, covtab, par)
    zbl_pad, tot_pad = _phase2(partial, ae_pad)
    return zbl_pad[:N_NODES], tot_pad[:N_NODES]


# speedup vs baseline: 449.4278x; 1.5325x over previous
import jax
import jax.numpy as jnp
import numpy as np
from jax import lax
from jax.experimental import pallas as pl
from jax.experimental.pallas import tpu as pltpu
from jax.experimental.pallas import tpu_sc as plsc

N_NODES = 100000
N_EDGES = 6400000
KE = 14.399645351950548
_COV = np.linspace(0.2, 2.6, 119).astype(np.float32)

NC, NS, L = 2, 16, 16
NW = NC * NS
ROW_W = 128
ROWS = N_EDGES // ROW_W
CHUNK_ROWS = 8
ROWS_PER_W = 1568
NPAIR_FULL = ROWS_PER_W // (2 * CHUNK_ROWS)
NPAIR_LAST = (ROWS - (NW - 1) * ROWS_PER_W) // (2 * CHUNK_ROWS)
ACC_PAD = 100352
SLICE = ACC_PAD // NS
P2W = ACC_PAD // NW


def _edge_body(an_h, ii_h, ij_h, len_h, pt_h, ct_h, par_h, out_h,
               an_v, pt_v, ct_v, par_v,
               ii_a, ij_a, len_a, val_a, ii_b, ij_b, len_b, val_b,
               stage_v, acc_sh, sem_a, sem_b, sem_s):
    cid = lax.axis_index("c")
    sid = lax.axis_index("s")
    w = cid * NS + sid

    pltpu.sync_copy(an_h, an_v)
    pltpu.sync_copy(pt_h, pt_v)
    pltpu.sync_copy(ct_h, ct_v)
    pltpu.sync_copy(par_h, par_v)

    zero = jnp.zeros((L,), jnp.float32)

    def zbody(i, carry):
        stage_v[pl.ds(i * L, L)] = zero
        return carry

    lax.fori_loop(0, SLICE // L, zbody, 0)
    pltpu.sync_copy(stage_v, acc_sh.at[pl.ds(sid * SLICE, SLICE)])
    plsc.subcore_barrier()

    c0 = par_v[0]
    c1 = par_v[1]
    c2 = par_v[2]
    c3 = par_v[3]
    nd0 = par_v[4]
    nd1 = par_v[5]
    nd2 = par_v[6]
    nd3 = par_v[7]
    ke2 = par_v[8]
    inv_zl = par_v[9]
    one = jnp.ones((L,), jnp.float32)
    zerov = jnp.zeros((L,), jnp.float32)

    base_row = w * ROWS_PER_W
    n_pair = jnp.where(w == NW - 1, NPAIR_LAST, NPAIR_FULL)

    def start_in(row0, ii_v, ij_v, len_v, sem):
        pltpu.async_copy(ii_h.at[pl.ds(row0, CHUNK_ROWS)], ii_v, sem)
        pltpu.async_copy(ij_h.at[pl.ds(row0, CHUNK_ROWS)], ij_v, sem)
        pltpu.async_copy(len_h.at[pl.ds(row0, CHUNK_ROWS)], len_v, sem)

    def wait_in(row0, ii_v, ij_v, len_v, sem):
        pltpu.make_async_copy(ii_h.at[pl.ds(row0, CHUNK_ROWS)], ii_v, sem).wait()
        pltpu.make_async_copy(ij_h.at[pl.ds(row0, CHUNK_ROWS)], ij_v, sem).wait()
        pltpu.make_async_copy(len_h.at[pl.ds(row0, CHUNK_ROWS)], len_v, sem).wait()

    def process(ii_v, ij_v, len_v, val_v):
        for r in range(CHUNK_ROWS):
            def vbody(v, c2_):
                sl = pl.ds(v * L, L)
                ii = ii_v[r, sl]
                ij = ij_v[r, sl]
                ln = len_v[r, sl]
                zi = plsc.load_gather(an_v, [ii])
                zj = plsc.load_gather(an_v, [ij])
                pi = plsc.load_gather(pt_v, [zi])
                pj = plsc.load_gather(pt_v, [zj])
                ci = plsc.load_gather(ct_v, [zi])
                cj = plsc.load_gather(ct_v, [zj])
                zif = zi.astype(jnp.float32)
                zjf = zj.astype(jnp.float32)
                t = ln * (pi + pj) * inv_zl
                f = (c0 * jnp.exp(nd0 * t) + c1 * jnp.exp(nd1 * t)
                     + c2 * jnp.exp(nd2 * t) + c3 * jnp.exp(nd3 * t))
                pref = ke2 * zif * zjf / ln
                x = ln / (ci + cj)
                x2 = x * x
                x6 = x2 * x2 * x2
                poly = one + x6 * (x * 48.0 - x2 * 21.0 - 28.0)
                val = jnp.where(x < one, pref * f * poly, zerov)
                val_v[r, sl] = val
                return c2_

            lax.fori_loop(0, ROW_W // L, vbody, 0)
            pltpu.async_copy(val_v.at[r], acc_sh.at[ii_v.at[r]], sem_s,
                             add=True)
        for r in range(CHUNK_ROWS):
            pltpu.make_async_copy(val_v.at[r], acc_sh.at[ii_v.at[r]],
                                  sem_s).wait()

    start_in(base_row, ii_a, ij_a, len_a, sem_a)

    def pair_body(p, carry):
        row_a = base_row + p * (2 * CHUNK_ROWS)
        row_b = row_a + CHUNK_ROWS
        start_in(row_b, ii_b, ij_b, len_b, sem_b)
        wait_in(row_a, ii_a, ij_a, len_a, sem_a)
        process(ii_a, ij_a, len_a, val_a)

        @pl.when(p + 1 < n_pair)
        def _():
            start_in(row_b + CHUNK_ROWS, ii_a, ij_a, len_a, sem_a)

        wait_in(row_b, ii_b, ij_b, len_b, sem_b)
        process(ii_b, ij_b, len_b, val_b)
        return carry

    lax.fori_loop(0, n_pair, pair_body, 0)

    plsc.subcore_barrier()
    pltpu.sync_copy(acc_sh.at[pl.ds(sid * SLICE, SLICE)], stage_v)
    pltpu.sync_copy(stage_v, out_h.at[pl.ds(cid * ACC_PAD + sid * SLICE, SLICE)])


def _reduce_body(p_h, ae_h, zbl_h, tot_h, a_v, b_v, e_v):
    cid = lax.axis_index("c")
    sid = lax.axis_index("s")
    w = cid * NS + sid
    off = w * P2W
    pltpu.sync_copy(p_h.at[pl.ds(off, P2W)], a_v)
    pltpu.sync_copy(p_h.at[pl.ds(ACC_PAD + off, P2W)], b_v)
    pltpu.sync_copy(ae_h.at[pl.ds(off, P2W)], e_v)

    def vb(v, carry):
        sl = pl.ds(v * L, L)
        z = a_v[sl] + b_v[sl]
        a_v[sl] = z
        e_v[sl] = e_v[sl] + z
        return carry

    lax.fori_loop(0, P2W // L, vb, 0)
    pltpu.sync_copy(a_v, zbl_h.at[pl.ds(off, P2W)])
    pltpu.sync_copy(e_v, tot_h.at[pl.ds(off, P2W)])


_MESH = plsc.VectorSubcoreMesh(core_axis_name="c", subcore_axis_name="s",
                               num_cores=NC, num_subcores=NS)
_CPARAMS = pltpu.CompilerParams(needs_layout_passes=False)

_phase1 = pl.kernel(
    _edge_body,
    out_type=jax.ShapeDtypeStruct((NC * ACC_PAD,), jnp.float32),
    mesh=_MESH,
    scratch_types=[
        pltpu.VMEM((ACC_PAD,), jnp.int32),
        pltpu.VMEM((128,), jnp.float32),
        pltpu.VMEM((128,), jnp.float32),
        pltpu.VMEM((16, L), jnp.float32),
        pltpu.VMEM((CHUNK_ROWS, ROW_W), jnp.int32),
        pltpu.VMEM((CHUNK_ROWS, ROW_W), jnp.int32),
        pltpu.VMEM((CHUNK_ROWS, ROW_W), jnp.float32),
        pltpu.VMEM((CHUNK_ROWS, ROW_W), jnp.float32),
        pltpu.VMEM((CHUNK_ROWS, ROW_W), jnp.int32),
        pltpu.VMEM((CHUNK_ROWS, ROW_W), jnp.int32),
        pltpu.VMEM((CHUNK_ROWS, ROW_W), jnp.float32),
        pltpu.VMEM((CHUNK_ROWS, ROW_W), jnp.float32),
        pltpu.VMEM((SLICE,), jnp.float32),
        pltpu.VMEM_SHARED((ACC_PAD,), jnp.float32),
        pltpu.SemaphoreType.DMA,
        pltpu.SemaphoreType.DMA,
        pltpu.SemaphoreType.DMA,
    ],
    compiler_params=_CPARAMS,
)

_phase2 = pl.kernel(
    _reduce_body,
    out_type=(jax.ShapeDtypeStruct((ACC_PAD,), jnp.float32),
              jax.ShapeDtypeStruct((ACC_PAD,), jnp.float32)),
    mesh=_MESH,
    scratch_types=[
        pltpu.VMEM((P2W,), jnp.float32),
        pltpu.VMEM((P2W,), jnp.float32),
        pltpu.VMEM((P2W,), jnp.float32),
    ],
    compiler_params=_CPARAMS,
)


def kernel(atomic_numbers, edge_index, lengths, atomic_energies,
           c_zbl_raw, d_zbl_raw, zbl_pow_raw, zbl_length_raw):
    an = jnp.zeros((ACC_PAD,), jnp.int32).at[:N_NODES].set(
        atomic_numbers.astype(jnp.int32))
    ii = edge_index[0].astype(jnp.int32).reshape(ROWS, ROW_W)
    ij = edge_index[1].astype(jnp.int32).reshape(ROWS, ROW_W)
    ln = lengths.reshape(ROWS, ROW_W).astype(jnp.float32)

    c = jax.nn.softplus(c_zbl_raw)
    c = c / c.sum()
    d = jax.nn.softplus(d_zbl_raw)
    p = jax.nn.softplus(zbl_pow_raw)[0]
    zl = jax.nn.softplus(zbl_length_raw)[0]

    powtab = jnp.arange(128, dtype=jnp.float32) ** p
    covtab = jnp.zeros((128,), jnp.float32).at[:119].set(jnp.asarray(_COV))
    scal = jnp.stack([c[0], c[1], c[2], c[3], -d[0], -d[1], -d[2], -d[3],
                      jnp.float32(KE / 2.0), 1.0 / zl,
                      jnp.float32(0.0), jnp.float32(0.0), jnp.float32(0.0),
                      jnp.float32(0.0), jnp.float32(0.0), jnp.float32(0.0)])
    par = jnp.repeat(scal[:, None], L, axis=1)

    ae_pad = jnp.zeros((ACC_PAD,), jnp.float32).at[:N_NODES].set(atomic_energies)

    partial = _phase1(an, ii, ij, ln, powtab, covtab, par)
    zbl_pad, tot_pad = _phase2(partial, ae_pad)
    return zbl_pad[:N_NODES], tot_pad[:N_NODES]


# speedup vs baseline: 757.1035x; 1.6846x over previous
import jax
import jax.numpy as jnp
import numpy as np
from jax import lax
from jax.experimental import pallas as pl
from jax.experimental.pallas import tpu as pltpu
from jax.experimental.pallas import tpu_sc as plsc

N_NODES = 100000
N_EDGES = 6400000
KE = 14.399645351950548
_COV = np.linspace(0.2, 2.6, 119).astype(np.float32)

NC, NS, L = 2, 16, 16
NW = NC * NS
ROW_W = 128
ROWS = N_EDGES // ROW_W
CHUNK_ROWS = 8
ROWS_PER_W = 1568
NPAIR_FULL = ROWS_PER_W // (2 * CHUNK_ROWS)
NPAIR_LAST = (ROWS - (NW - 1) * ROWS_PER_W) // (2 * CHUNK_ROWS)
ACC_PAD = 100352
SLICE = ACC_PAD // NS
P2W = ACC_PAD // NW


def _edge_body(an_h, ii_h, ij_h, len_h, pt_h, ct_h, par_h, out_h,
               an_v, pt_v, ct_v, par_v,
               ii_a, ij_a, len_a, val_a, ii_b, ij_b, len_b, val_b,
               stage_v, acc_sh, sem_a, sem_b, sem_s):
    cid = lax.axis_index("c")
    sid = lax.axis_index("s")
    w = cid * NS + sid

    pltpu.sync_copy(an_h, an_v)
    pltpu.sync_copy(pt_h, pt_v)
    pltpu.sync_copy(ct_h, ct_v)
    pltpu.sync_copy(par_h, par_v)

    zero = jnp.zeros((L,), jnp.float32)

    def zbody(i, carry):
        stage_v[pl.ds(i * L, L)] = zero
        return carry

    lax.fori_loop(0, SLICE // L, zbody, 0)
    pltpu.sync_copy(stage_v, acc_sh.at[pl.ds(sid * SLICE, SLICE)])
    plsc.subcore_barrier()

    c0 = par_v[0]
    c1 = par_v[1]
    c2 = par_v[2]
    c3 = par_v[3]
    nd0 = par_v[4]
    nd1 = par_v[5]
    nd2 = par_v[6]
    nd3 = par_v[7]
    ke2 = par_v[8]
    inv_zl = par_v[9]
    one = jnp.ones((L,), jnp.float32)
    zerov = jnp.zeros((L,), jnp.float32)

    base_row = w * ROWS_PER_W
    n_pair = jnp.where(w == NW - 1, NPAIR_LAST, NPAIR_FULL)

    def start_in(row0, ii_v, ij_v, len_v, sem):
        pltpu.async_copy(ii_h.at[pl.ds(row0, CHUNK_ROWS)], ii_v, sem)
        pltpu.async_copy(ij_h.at[pl.ds(row0, CHUNK_ROWS)], ij_v, sem)
        pltpu.async_copy(len_h.at[pl.ds(row0, CHUNK_ROWS)], len_v, sem)

    def wait_in(row0, ii_v, ij_v, len_v, sem):
        pltpu.make_async_copy(ii_h.at[pl.ds(row0, CHUNK_ROWS)], ii_v, sem).wait()
        pltpu.make_async_copy(ij_h.at[pl.ds(row0, CHUNK_ROWS)], ij_v, sem).wait()
        pltpu.make_async_copy(len_h.at[pl.ds(row0, CHUNK_ROWS)], len_v, sem).wait()

    def process(ii_v, ij_v, len_v, val_v):
        def row_body(r, carry):
            def vbody(v):
                sl = pl.ds(v * L, L)
                ii = ii_v[r, sl]
                ij = ij_v[r, sl]
                ln = len_v[r, sl]
                zi = plsc.load_gather(an_v, [ii])
                zj = plsc.load_gather(an_v, [ij])
                pi = plsc.load_gather(pt_v, [zi])
                pj = plsc.load_gather(pt_v, [zj])
                ci = plsc.load_gather(ct_v, [zi])
                cj = plsc.load_gather(ct_v, [zj])
                zif = zi.astype(jnp.float32)
                zjf = zj.astype(jnp.float32)
                t = ln * (pi + pj)
                f = (c0 * jnp.exp(nd0 * t) + c1 * jnp.exp(nd1 * t)
                     + c2 * jnp.exp(nd2 * t) + c3 * jnp.exp(nd3 * t))
                pref = zif * zjf / ln
                x = ln / (ci + cj)
                x2 = x * x
                x6 = x2 * x2 * x2
                poly = one + x6 * (x * 48.0 - x2 * 21.0 - 28.0)
                val = jnp.where(x < one, pref * f * poly, zerov)
                val_v[r, sl] = val

            plsc.parallel_loop(0, ROW_W // L, 1, unroll=4)(vbody)
            pltpu.async_copy(val_v.at[r], acc_sh.at[ii_v.at[r]], sem_s,
                             add=True)
            return carry

        lax.fori_loop(0, CHUNK_ROWS, row_body, 0)

        def drain_body(r, carry):
            pltpu.make_async_copy(val_v.at[r], acc_sh.at[ii_v.at[r]],
                                  sem_s).wait()
            return carry

        lax.fori_loop(0, CHUNK_ROWS, drain_body, 0)

    start_in(base_row, ii_a, ij_a, len_a, sem_a)

    def pair_body(p, carry):
        row_a = base_row + p * (2 * CHUNK_ROWS)
        row_b = row_a + CHUNK_ROWS
        start_in(row_b, ii_b, ij_b, len_b, sem_b)
        wait_in(row_a, ii_a, ij_a, len_a, sem_a)
        process(ii_a, ij_a, len_a, val_a)

        @pl.when(p + 1 < n_pair)
        def _():
            start_in(row_b + CHUNK_ROWS, ii_a, ij_a, len_a, sem_a)

        wait_in(row_b, ii_b, ij_b, len_b, sem_b)
        process(ii_b, ij_b, len_b, val_b)
        return carry

    lax.fori_loop(0, n_pair, pair_body, 0)

    plsc.subcore_barrier()
    pltpu.sync_copy(acc_sh.at[pl.ds(sid * SLICE, SLICE)], stage_v)
    pltpu.sync_copy(stage_v, out_h.at[pl.ds(cid * ACC_PAD + sid * SLICE, SLICE)])


def _reduce_body(p_h, ae_h, zbl_h, tot_h, a_v, b_v, e_v):
    cid = lax.axis_index("c")
    sid = lax.axis_index("s")
    w = cid * NS + sid
    off = w * P2W
    pltpu.sync_copy(p_h.at[pl.ds(off, P2W)], a_v)
    pltpu.sync_copy(p_h.at[pl.ds(ACC_PAD + off, P2W)], b_v)
    pltpu.sync_copy(ae_h.at[pl.ds(off, P2W)], e_v)

    def vb(v, carry):
        sl = pl.ds(v * L, L)
        z = a_v[sl] + b_v[sl]
        a_v[sl] = z
        e_v[sl] = e_v[sl] + z
        return carry

    lax.fori_loop(0, P2W // L, vb, 0)
    pltpu.sync_copy(a_v, zbl_h.at[pl.ds(off, P2W)])
    pltpu.sync_copy(e_v, tot_h.at[pl.ds(off, P2W)])


_MESH = plsc.VectorSubcoreMesh(core_axis_name="c", subcore_axis_name="s",
                               num_cores=NC, num_subcores=NS)
_CPARAMS = pltpu.CompilerParams(needs_layout_passes=False)

_phase1 = pl.kernel(
    _edge_body,
    out_type=jax.ShapeDtypeStruct((NC * ACC_PAD,), jnp.float32),
    mesh=_MESH,
    scratch_types=[
        pltpu.VMEM((ACC_PAD,), jnp.int32),
        pltpu.VMEM((128,), jnp.float32),
        pltpu.VMEM((128,), jnp.float32),
        pltpu.VMEM((16, L), jnp.float32),
        pltpu.VMEM((CHUNK_ROWS, ROW_W), jnp.int32),
        pltpu.VMEM((CHUNK_ROWS, ROW_W), jnp.int32),
        pltpu.VMEM((CHUNK_ROWS, ROW_W), jnp.float32),
        pltpu.VMEM((CHUNK_ROWS, ROW_W), jnp.float32),
        pltpu.VMEM((CHUNK_ROWS, ROW_W), jnp.int32),
        pltpu.VMEM((CHUNK_ROWS, ROW_W), jnp.int32),
        pltpu.VMEM((CHUNK_ROWS, ROW_W), jnp.float32),
        pltpu.VMEM((CHUNK_ROWS, ROW_W), jnp.float32),
        pltpu.VMEM((SLICE,), jnp.float32),
        pltpu.VMEM_SHARED((ACC_PAD,), jnp.float32),
        pltpu.SemaphoreType.DMA,
        pltpu.SemaphoreType.DMA,
        pltpu.SemaphoreType.DMA,
    ],
    compiler_params=_CPARAMS,
)

_phase2 = pl.kernel(
    _reduce_body,
    out_type=(jax.ShapeDtypeStruct((ACC_PAD,), jnp.float32),
              jax.ShapeDtypeStruct((ACC_PAD,), jnp.float32)),
    mesh=_MESH,
    scratch_types=[
        pltpu.VMEM((P2W,), jnp.float32),
        pltpu.VMEM((P2W,), jnp.float32),
        pltpu.VMEM((P2W,), jnp.float32),
    ],
    compiler_params=_CPARAMS,
)


def kernel(atomic_numbers, edge_index, lengths, atomic_energies,
           c_zbl_raw, d_zbl_raw, zbl_pow_raw, zbl_length_raw):
    an = jnp.zeros((ACC_PAD,), jnp.int32).at[:N_NODES].set(
        atomic_numbers.astype(jnp.int32))
    ii = edge_index[0].astype(jnp.int32).reshape(ROWS, ROW_W)
    ij = edge_index[1].astype(jnp.int32).reshape(ROWS, ROW_W)
    ln = lengths.reshape(ROWS, ROW_W).astype(jnp.float32)

    c = jax.nn.softplus(c_zbl_raw)
    c = c / c.sum()
    d = jax.nn.softplus(d_zbl_raw)
    p = jax.nn.softplus(zbl_pow_raw)[0]
    zl = jax.nn.softplus(zbl_length_raw)[0]

    powtab = (jnp.arange(128, dtype=jnp.float32) ** p) / zl
    covtab = jnp.zeros((128,), jnp.float32).at[:119].set(jnp.asarray(_COV))
    c = c * jnp.float32(KE / 2.0)
    scal = jnp.stack([c[0], c[1], c[2], c[3], -d[0], -d[1], -d[2], -d[3],
                      jnp.float32(KE / 2.0), 1.0 / zl,
                      jnp.float32(0.0), jnp.float32(0.0), jnp.float32(0.0),
                      jnp.float32(0.0), jnp.float32(0.0), jnp.float32(0.0)])
    par = jnp.repeat(scal[:, None], L, axis=1)

    ae_pad = jnp.zeros((ACC_PAD,), jnp.float32).at[:N_NODES].set(atomic_energies)

    partial = _phase1(an, ii, ij, ln, powtab, covtab, par)
    zbl_pad, tot_pad = _phase2(partial, ae_pad)
    return zbl_pad[:N_NODES], tot_pad[:N_NODES]


# speedup vs baseline: 950.2161x; 1.2551x over previous
import jax
import jax.numpy as jnp
import numpy as np
from jax import lax
from jax.experimental import pallas as pl
from jax.experimental.pallas import tpu as pltpu
from jax.experimental.pallas import tpu_sc as plsc

N_NODES = 100000
N_EDGES = 6400000
KE = 14.399645351950548
_COV = np.linspace(0.2, 2.6, 119).astype(np.float32)

NC, NS, L = 2, 16, 16
NW = NC * NS
ROW_W = 128
ROWS = N_EDGES // ROW_W
CHUNK_ROWS = 8
ROWS_PER_W = 1568
NPAIR_FULL = ROWS_PER_W // (2 * CHUNK_ROWS)
NPAIR_LAST = (ROWS - (NW - 1) * ROWS_PER_W) // (2 * CHUNK_ROWS)
ACC_PAD = 100352
SLICE = ACC_PAD // NS
P2W = ACC_PAD // NW


def _edge_body(an_h, ei_h, len_h, pt_h, ct_h, par_h, out_h,
               an_v, pt_v, ct_v, par_v,
               ii_a, ij_a, len_a, val_a, ii_b, ij_b, len_b, val_b,
               stage_v, acc_sh, sem_a, sem_b, sem_s):
    cid = lax.axis_index("c")
    sid = lax.axis_index("s")
    w = cid * NS + sid

    pltpu.sync_copy(an_h, an_v)
    pltpu.sync_copy(pt_h, pt_v)
    pltpu.sync_copy(ct_h, ct_v)
    pltpu.sync_copy(par_h, par_v)

    zero = jnp.zeros((L,), jnp.float32)

    def zbody(i, carry):
        stage_v[pl.ds(i * L, L)] = zero
        return carry

    lax.fori_loop(0, SLICE // L, zbody, 0)
    pltpu.sync_copy(stage_v, acc_sh.at[pl.ds(sid * SLICE, SLICE)])
    plsc.subcore_barrier()

    c0 = par_v[0]
    c1 = par_v[1]
    c2 = par_v[2]
    c3 = par_v[3]
    nd0 = par_v[4]
    nd1 = par_v[5]
    nd2 = par_v[6]
    nd3 = par_v[7]
    ke2 = par_v[8]
    inv_zl = par_v[9]
    one = jnp.ones((L,), jnp.float32)
    zerov = jnp.zeros((L,), jnp.float32)

    base_row = w * ROWS_PER_W
    n_pair = jnp.where(w == NW - 1, NPAIR_LAST, NPAIR_FULL)

    def start_in(row0, ii_v, ij_v, len_v, sem):
        pltpu.async_copy(ei_h.at[0].at[pl.ds(row0, CHUNK_ROWS)], ii_v, sem)
        pltpu.async_copy(ei_h.at[1].at[pl.ds(row0, CHUNK_ROWS)], ij_v, sem)
        pltpu.async_copy(len_h.at[pl.ds(row0, CHUNK_ROWS)], len_v, sem)

    def wait_in(row0, ii_v, ij_v, len_v, sem):
        pltpu.make_async_copy(ei_h.at[0].at[pl.ds(row0, CHUNK_ROWS)], ii_v, sem).wait()
        pltpu.make_async_copy(ei_h.at[1].at[pl.ds(row0, CHUNK_ROWS)], ij_v, sem).wait()
        pltpu.make_async_copy(len_h.at[pl.ds(row0, CHUNK_ROWS)], len_v, sem).wait()

    def process(ii_v, ij_v, len_v, val_v):
        def row_body(r, carry):
            def vbody(v):
                sl = pl.ds(v * L, L)
                ii = ii_v[r, sl]
                ij = ij_v[r, sl]
                ln = len_v[r, sl]
                zi = plsc.load_gather(an_v, [ii])
                zj = plsc.load_gather(an_v, [ij])
                pi = plsc.load_gather(pt_v, [zi])
                pj = plsc.load_gather(pt_v, [zj])
                ci = plsc.load_gather(ct_v, [zi])
                cj = plsc.load_gather(ct_v, [zj])
                zif = zi.astype(jnp.float32)
                zjf = zj.astype(jnp.float32)
                t = ln * (pi + pj)
                f = (c0 * jnp.exp(nd0 * t) + c1 * jnp.exp(nd1 * t)
                     + c2 * jnp.exp(nd2 * t) + c3 * jnp.exp(nd3 * t))
                pref = zif * zjf / ln
                x = ln / (ci + cj)
                x2 = x * x
                x6 = x2 * x2 * x2
                poly = one + x6 * (x * 48.0 - x2 * 21.0 - 28.0)
                val = jnp.where(x < one, pref * f * poly, zerov)
                val_v[r, sl] = val

            plsc.parallel_loop(0, ROW_W // L, 1, unroll=8)(vbody)
            pltpu.async_copy(val_v.at[r], acc_sh.at[ii_v.at[r]], sem_s,
                             add=True)
            return carry

        lax.fori_loop(0, CHUNK_ROWS, row_body, 0)

        def drain_body(r, carry):
            pltpu.make_async_copy(val_v.at[r], acc_sh.at[ii_v.at[r]],
                                  sem_s).wait()
            return carry

        lax.fori_loop(0, CHUNK_ROWS, drain_body, 0)

    start_in(base_row, ii_a, ij_a, len_a, sem_a)

    def pair_body(p, carry):
        row_a = base_row + p * (2 * CHUNK_ROWS)
        row_b = row_a + CHUNK_ROWS
        start_in(row_b, ii_b, ij_b, len_b, sem_b)
        wait_in(row_a, ii_a, ij_a, len_a, sem_a)
        process(ii_a, ij_a, len_a, val_a)

        @pl.when(p + 1 < n_pair)
        def _():
            start_in(row_b + CHUNK_ROWS, ii_a, ij_a, len_a, sem_a)

        wait_in(row_b, ii_b, ij_b, len_b, sem_b)
        process(ii_b, ij_b, len_b, val_b)
        return carry

    lax.fori_loop(0, n_pair, pair_body, 0)

    plsc.subcore_barrier()
    pltpu.sync_copy(acc_sh.at[pl.ds(sid * SLICE, SLICE)], stage_v)
    pltpu.sync_copy(stage_v, out_h.at[pl.ds(cid * ACC_PAD + sid * SLICE, SLICE)])


def _reduce_body(p_h, ae_h, zbl_h, tot_h, a_v, b_v, e_v):
    cid = lax.axis_index("c")
    sid = lax.axis_index("s")
    w = cid * NS + sid
    off = w * P2W
    pltpu.sync_copy(p_h.at[pl.ds(off, P2W)], a_v)
    pltpu.sync_copy(p_h.at[pl.ds(ACC_PAD + off, P2W)], b_v)
    pltpu.sync_copy(ae_h.at[pl.ds(off, P2W)], e_v)

    def vb(v, carry):
        sl = pl.ds(v * L, L)
        z = a_v[sl] + b_v[sl]
        a_v[sl] = z
        e_v[sl] = e_v[sl] + z
        return carry

    lax.fori_loop(0, P2W // L, vb, 0)
    pltpu.sync_copy(a_v, zbl_h.at[pl.ds(off, P2W)])
    pltpu.sync_copy(e_v, tot_h.at[pl.ds(off, P2W)])


_MESH = plsc.VectorSubcoreMesh(core_axis_name="c", subcore_axis_name="s",
                               num_cores=NC, num_subcores=NS)
_CPARAMS = pltpu.CompilerParams(needs_layout_passes=False)

_phase1 = pl.kernel(
    _edge_body,
    out_type=jax.ShapeDtypeStruct((NC * ACC_PAD,), jnp.float32),
    mesh=_MESH,
    scratch_types=[
        pltpu.VMEM((ACC_PAD,), jnp.int32),
        pltpu.VMEM((128,), jnp.float32),
        pltpu.VMEM((128,), jnp.float32),
        pltpu.VMEM((16, L), jnp.float32),
        pltpu.VMEM((CHUNK_ROWS, ROW_W), jnp.int32),
        pltpu.VMEM((CHUNK_ROWS, ROW_W), jnp.int32),
        pltpu.VMEM((CHUNK_ROWS, ROW_W), jnp.float32),
        pltpu.VMEM((CHUNK_ROWS, ROW_W), jnp.float32),
        pltpu.VMEM((CHUNK_ROWS, ROW_W), jnp.int32),
        pltpu.VMEM((CHUNK_ROWS, ROW_W), jnp.int32),
        pltpu.VMEM((CHUNK_ROWS, ROW_W), jnp.float32),
        pltpu.VMEM((CHUNK_ROWS, ROW_W), jnp.float32),
        pltpu.VMEM((SLICE,), jnp.float32),
        pltpu.VMEM_SHARED((ACC_PAD,), jnp.float32),
        pltpu.SemaphoreType.DMA,
        pltpu.SemaphoreType.DMA,
        pltpu.SemaphoreType.DMA,
    ],
    compiler_params=_CPARAMS,
)

_phase2 = pl.kernel(
    _reduce_body,
    out_type=(jax.ShapeDtypeStruct((ACC_PAD,), jnp.float32),
              jax.ShapeDtypeStruct((ACC_PAD,), jnp.float32)),
    mesh=_MESH,
    scratch_types=[
        pltpu.VMEM((P2W,), jnp.float32),
        pltpu.VMEM((P2W,), jnp.float32),
        pltpu.VMEM((P2W,), jnp.float32),
    ],
    compiler_params=_CPARAMS,
)


def kernel(atomic_numbers, edge_index, lengths, atomic_energies,
           c_zbl_raw, d_zbl_raw, zbl_pow_raw, zbl_length_raw):
    an = jnp.zeros((ACC_PAD,), jnp.int32).at[:N_NODES].set(
        atomic_numbers.astype(jnp.int32))
    ei = edge_index.astype(jnp.int32).reshape(2, ROWS, ROW_W)
    ln = lengths.reshape(ROWS, ROW_W).astype(jnp.float32)

    c = jax.nn.softplus(c_zbl_raw)
    c = c / c.sum()
    d = jax.nn.softplus(d_zbl_raw)
    p = jax.nn.softplus(zbl_pow_raw)[0]
    zl = jax.nn.softplus(zbl_length_raw)[0]

    powtab = (jnp.arange(128, dtype=jnp.float32) ** p) / zl
    covtab = jnp.zeros((128,), jnp.float32).at[:119].set(jnp.asarray(_COV))
    c = c * jnp.float32(KE / 2.0)
    scal = jnp.stack([c[0], c[1], c[2], c[3], -d[0], -d[1], -d[2], -d[3],
                      jnp.float32(KE / 2.0), 1.0 / zl,
                      jnp.float32(0.0), jnp.float32(0.0), jnp.float32(0.0),
                      jnp.float32(0.0), jnp.float32(0.0), jnp.float32(0.0)])
    par = jnp.repeat(scal[:, None], L, axis=1)

    ae_pad = jnp.zeros((ACC_PAD,), jnp.float32).at[:N_NODES].set(atomic_energies)

    partial = _phase1(an, ei, ln, powtab, covtab, par)
    zbl_pad, tot_pad = _phase2(partial, ae_pad)
    return zbl_pad[:N_NODES], tot_pad[:N_NODES]


# speedup vs baseline: 1050.5669x; 1.1056x over previous
import jax
import jax.numpy as jnp
import numpy as np
from jax import lax
from jax.experimental import pallas as pl
from jax.experimental.pallas import tpu as pltpu
from jax.experimental.pallas import tpu_sc as plsc

N_NODES = 100000
N_EDGES = 6400000
KE = 14.399645351950548
_COV = np.linspace(0.2, 2.6, 119).astype(np.float32)

NC, NS, L = 2, 16, 16
NW = NC * NS
ROW_W = 128
ROWS = N_EDGES // ROW_W
CHUNK_ROWS = 8
CHUNK_EDGES = CHUNK_ROWS * 128
ROWS_PER_W = 1568
NPAIR_FULL = ROWS_PER_W // (2 * CHUNK_ROWS)
NPAIR_LAST = (ROWS - (NW - 1) * ROWS_PER_W) // (2 * CHUNK_ROWS)
ACC_PAD = 100352
SLICE = ACC_PAD // NS
P2W = ACC_PAD // NW


def _edge_body(an_h, ei_h, len_h, pt_h, ct_h, par_h, out_h,
               an_v, pt_v, ct_v, par_v,
               ei_a, len_a, ei_b, len_b, ii2_v, val_v,
               stage_v, acc_sh, sem_a, sem_b, sem_s):
    cid = lax.axis_index("c")
    sid = lax.axis_index("s")
    w = cid * NS + sid

    pltpu.sync_copy(an_h, an_v)
    pltpu.sync_copy(pt_h, pt_v)
    pltpu.sync_copy(ct_h, ct_v)
    pltpu.sync_copy(par_h, par_v)

    zero = jnp.zeros((L,), jnp.float32)

    def zbody(i, carry):
        stage_v[pl.ds(i * L, L)] = zero
        return carry

    lax.fori_loop(0, SLICE // L, zbody, 0)
    pltpu.sync_copy(stage_v, acc_sh.at[pl.ds(sid * SLICE, SLICE)])
    plsc.subcore_barrier()

    c0 = par_v[0]
    c1 = par_v[1]
    c2 = par_v[2]
    c3 = par_v[3]
    nd0 = par_v[4]
    nd1 = par_v[5]
    nd2 = par_v[6]
    nd3 = par_v[7]
    ke2 = par_v[8]
    inv_zl = par_v[9]
    one = jnp.ones((L,), jnp.float32)
    zerov = jnp.zeros((L,), jnp.float32)

    base_row = w * ROWS_PER_W
    n_pair = jnp.where(w == NW - 1, NPAIR_LAST, NPAIR_FULL)

    def start_in(row0, ei_v, len_v, sem):
        pltpu.async_copy(ei_h.at[:, pl.ds(row0 * ROW_W, CHUNK_EDGES)], ei_v, sem)
        pltpu.async_copy(len_h.at[pl.ds(row0, CHUNK_ROWS)], len_v, sem)

    def wait_in(row0, ei_v, len_v, sem):
        pltpu.make_async_copy(ei_h.at[:, pl.ds(row0 * ROW_W, CHUNK_EDGES)], ei_v, sem).wait()
        pltpu.make_async_copy(len_h.at[pl.ds(row0, CHUNK_ROWS)], len_v, sem).wait()

    def process(ei_v, len_v):
        def row_body(r, carry):
            def vbody(v):
                sl = pl.ds(v * L, L)
                esl = pl.ds(r * ROW_W + v * L, L)
                ii = ei_v[0, esl]
                ij = ei_v[1, esl]
                ln = len_v[r, sl]
                zi = plsc.load_gather(an_v, [ii])
                zj = plsc.load_gather(an_v, [ij])
                pi = plsc.load_gather(pt_v, [zi])
                pj = plsc.load_gather(pt_v, [zj])
                ci = plsc.load_gather(ct_v, [zi])
                cj = plsc.load_gather(ct_v, [zj])
                zif = zi.astype(jnp.float32)
                zjf = zj.astype(jnp.float32)
                t = ln * (pi + pj)
                f = (c0 * jnp.exp(nd0 * t) + c1 * jnp.exp(nd1 * t)
                     + c2 * jnp.exp(nd2 * t) + c3 * jnp.exp(nd3 * t))
                pref = zif * zjf / ln
                x = ln / (ci + cj)
                x2 = x * x
                x6 = x2 * x2 * x2
                poly = one + x6 * (x * 48.0 - x2 * 21.0 - 28.0)
                val = jnp.where(x < one, pref * f * poly, zerov)
                val_v[r, sl] = val
                ii2_v[r, sl] = ii

            plsc.parallel_loop(0, ROW_W // L, 1, unroll=8)(vbody)
            pltpu.async_copy(val_v.at[r], acc_sh.at[ii2_v.at[r]], sem_s,
                             add=True)
            return carry

        lax.fori_loop(0, CHUNK_ROWS, row_body, 0)

        def drain_body(r, carry):
            pltpu.make_async_copy(val_v.at[r], acc_sh.at[ii2_v.at[r]],
                                  sem_s).wait()
            return carry

        lax.fori_loop(0, CHUNK_ROWS, drain_body, 0)

    start_in(base_row, ei_a, len_a, sem_a)

    def pair_body(p, carry):
        row_a = base_row + p * (2 * CHUNK_ROWS)
        row_b = row_a + CHUNK_ROWS
        start_in(row_b, ei_b, len_b, sem_b)
        wait_in(row_a, ei_a, len_a, sem_a)
        process(ei_a, len_a)

        @pl.when(p + 1 < n_pair)
        def _():
            start_in(row_b + CHUNK_ROWS, ei_a, len_a, sem_a)

        wait_in(row_b, ei_b, len_b, sem_b)
        process(ei_b, len_b)
        return carry

    lax.fori_loop(0, n_pair, pair_body, 0)

    plsc.subcore_barrier()
    pltpu.sync_copy(acc_sh.at[pl.ds(sid * SLICE, SLICE)], stage_v)
    pltpu.sync_copy(stage_v, out_h.at[pl.ds(cid * ACC_PAD + sid * SLICE, SLICE)])


def _reduce_body(p_h, ae_h, zbl_h, tot_h, a_v, b_v, e_v):
    cid = lax.axis_index("c")
    sid = lax.axis_index("s")
    w = cid * NS + sid
    off = w * P2W
    pltpu.sync_copy(p_h.at[pl.ds(off, P2W)], a_v)
    pltpu.sync_copy(p_h.at[pl.ds(ACC_PAD + off, P2W)], b_v)
    pltpu.sync_copy(ae_h.at[pl.ds(off, P2W)], e_v)

    def vb(v, carry):
        sl = pl.ds(v * L, L)
        z = a_v[sl] + b_v[sl]
        a_v[sl] = z
        e_v[sl] = e_v[sl] + z
        return carry

    lax.fori_loop(0, P2W // L, vb, 0)
    pltpu.sync_copy(a_v, zbl_h.at[pl.ds(off, P2W)])
    pltpu.sync_copy(e_v, tot_h.at[pl.ds(off, P2W)])


_MESH = plsc.VectorSubcoreMesh(core_axis_name="c", subcore_axis_name="s",
                               num_cores=NC, num_subcores=NS)
_CPARAMS = pltpu.CompilerParams(needs_layout_passes=False)

_phase1 = pl.kernel(
    _edge_body,
    out_type=jax.ShapeDtypeStruct((NC * ACC_PAD,), jnp.float32),
    mesh=_MESH,
    scratch_types=[
        pltpu.VMEM((ACC_PAD,), jnp.int32),
        pltpu.VMEM((128,), jnp.float32),
        pltpu.VMEM((128,), jnp.float32),
        pltpu.VMEM((16, L), jnp.float32),
        pltpu.VMEM((2, CHUNK_EDGES), jnp.int32),
        pltpu.VMEM((CHUNK_ROWS, ROW_W), jnp.float32),
        pltpu.VMEM((2, CHUNK_EDGES), jnp.int32),
        pltpu.VMEM((CHUNK_ROWS, ROW_W), jnp.float32),
        pltpu.VMEM((CHUNK_ROWS, ROW_W), jnp.int32),
        pltpu.VMEM((CHUNK_ROWS, ROW_W), jnp.float32),
        pltpu.VMEM((SLICE,), jnp.float32),
        pltpu.VMEM_SHARED((ACC_PAD,), jnp.float32),
        pltpu.SemaphoreType.DMA,
        pltpu.SemaphoreType.DMA,
        pltpu.SemaphoreType.DMA,
    ],
    compiler_params=_CPARAMS,
)

_phase2 = pl.kernel(
    _reduce_body,
    out_type=(jax.ShapeDtypeStruct((ACC_PAD,), jnp.float32),
              jax.ShapeDtypeStruct((ACC_PAD,), jnp.float32)),
    mesh=_MESH,
    scratch_types=[
        pltpu.VMEM((P2W,), jnp.float32),
        pltpu.VMEM((P2W,), jnp.float32),
        pltpu.VMEM((P2W,), jnp.float32),
    ],
    compiler_params=_CPARAMS,
)


def kernel(atomic_numbers, edge_index, lengths, atomic_energies,
           c_zbl_raw, d_zbl_raw, zbl_pow_raw, zbl_length_raw):
    an = jnp.zeros((ACC_PAD,), jnp.int32).at[:N_NODES].set(
        atomic_numbers.astype(jnp.int32))
    ei = edge_index.astype(jnp.int32)
    ln = lengths.reshape(ROWS, ROW_W).astype(jnp.float32)

    c = jax.nn.softplus(c_zbl_raw)
    c = c / c.sum()
    d = jax.nn.softplus(d_zbl_raw)
    p = jax.nn.softplus(zbl_pow_raw)[0]
    zl = jax.nn.softplus(zbl_length_raw)[0]

    powtab = (jnp.arange(128, dtype=jnp.float32) ** p) / zl
    covtab = jnp.zeros((128,), jnp.float32).at[:119].set(jnp.asarray(_COV))
    c = c * jnp.float32(KE / 2.0)
    scal = jnp.stack([c[0], c[1], c[2], c[3], -d[0], -d[1], -d[2], -d[3],
                      jnp.float32(KE / 2.0), 1.0 / zl,
                      jnp.float32(0.0), jnp.float32(0.0), jnp.float32(0.0),
                      jnp.float32(0.0), jnp.float32(0.0), jnp.float32(0.0)])
    par = jnp.repeat(scal[:, None], L, axis=1)

    ae_pad = jnp.zeros((ACC_PAD,), jnp.float32).at[:N_NODES].set(atomic_energies)

    partial = _phase1(an, ei, ln, powtab, covtab, par)
    zbl_pad, tot_pad = _phase2(partial, ae_pad)
    return zbl_pad[:N_NODES], tot_pad[:N_NODES]


# speedup vs baseline: 1235.0101x; 1.1756x over previous
import jax
import jax.numpy as jnp
import numpy as np
from jax import lax
from jax.experimental import pallas as pl
from jax.experimental.pallas import tpu as pltpu
from jax.experimental.pallas import tpu_sc as plsc

N_NODES = 100000
N_EDGES = 6400000
KE = 14.399645351950548
_COV = np.linspace(0.2, 2.6, 119).astype(np.float32)

NC, NS, L = 2, 16, 16
NW = NC * NS
ROW_W = 128
ROWS = N_EDGES // ROW_W
CHUNK_ROWS = 8
CHUNK_EDGES = CHUNK_ROWS * 128
ROWS_PER_W = 1568
NPAIR_FULL = ROWS_PER_W // (2 * CHUNK_ROWS)
NPAIR_LAST = (ROWS - (NW - 1) * ROWS_PER_W) // (2 * CHUNK_ROWS)
ACC_PAD = 100352
SLICE = ACC_PAD // NS
P2W = ACC_PAD // NW


FBINS = 2048
FTAB = FBINS + 128


def _edge_body(an_h, ei_h, len_h, pt_h, ct_h, ft_h, out_h,
               an_v, pt_v, ct_v, ft_v,
               ei_a, len_a, ei_b, len_b, ii2_v, val_v,
               stage_v, acc_sh, sem_a, sem_b, sem_s):
    cid = lax.axis_index("c")
    sid = lax.axis_index("s")
    w = cid * NS + sid

    pltpu.sync_copy(an_h, an_v)
    pltpu.sync_copy(pt_h, pt_v)
    pltpu.sync_copy(ct_h, ct_v)
    pltpu.sync_copy(ft_h, ft_v)

    zero = jnp.zeros((L,), jnp.float32)

    def zbody(i, carry):
        stage_v[pl.ds(i * L, L)] = zero
        return carry

    lax.fori_loop(0, SLICE // L, zbody, 0)
    pltpu.sync_copy(stage_v, acc_sh.at[pl.ds(sid * SLICE, SLICE)])
    plsc.subcore_barrier()

    one = jnp.ones((L,), jnp.float32)
    zerov = jnp.zeros((L,), jnp.float32)

    base_row = w * ROWS_PER_W
    n_pair = jnp.where(w == NW - 1, NPAIR_LAST, NPAIR_FULL)

    def start_in(row0, ei_v, len_v, sem):
        pltpu.async_copy(ei_h.at[:, pl.ds(row0 * ROW_W, CHUNK_EDGES)], ei_v, sem)
        pltpu.async_copy(len_h.at[pl.ds(row0, CHUNK_ROWS)], len_v, sem)

    def wait_in(row0, ei_v, len_v, sem):
        pltpu.make_async_copy(ei_h.at[:, pl.ds(row0 * ROW_W, CHUNK_EDGES)], ei_v, sem).wait()
        pltpu.make_async_copy(len_h.at[pl.ds(row0, CHUNK_ROWS)], len_v, sem).wait()

    def process(ei_v, len_v):
        def row_body(r, carry):
            def vbody(v):
                sl = pl.ds(v * L, L)
                esl = pl.ds(r * ROW_W + v * L, L)
                ii = ei_v[0, esl]
                ij = ei_v[1, esl]
                ln = len_v[r, sl]
                zi = plsc.load_gather(an_v, [ii])
                zj = plsc.load_gather(an_v, [ij])
                pi = plsc.load_gather(pt_v, [zi])
                pj = plsc.load_gather(pt_v, [zj])
                ci = plsc.load_gather(ct_v, [zi])
                cj = plsc.load_gather(ct_v, [zj])
                zif = zi.astype(jnp.float32)
                zjf = zj.astype(jnp.float32)
                u = ln * (pi + pj)
                k = u.astype(jnp.int32)
                frac = u - k.astype(jnp.float32)
                f0 = plsc.load_gather(ft_v, [k])
                f1 = plsc.load_gather(ft_v, [k + 1])
                f = f0 + frac * (f1 - f0)
                pref = zif * zjf / ln
                x = ln / (ci + cj)
                x2 = x * x
                x6 = x2 * x2 * x2
                poly = one + x6 * (x * 48.0 - x2 * 21.0 - 28.0)
                val = jnp.where(x < one, pref * f * poly, zerov)
                val_v[r, sl] = val
                ii2_v[r, sl] = ii

            plsc.parallel_loop(0, ROW_W // L, 1, unroll=8)(vbody)
            pltpu.async_copy(val_v.at[r], acc_sh.at[ii2_v.at[r]], sem_s,
                             add=True)
            return carry

        lax.fori_loop(0, CHUNK_ROWS, row_body, 0)

        def drain_body(r, carry):
            pltpu.make_async_copy(val_v.at[r], acc_sh.at[ii2_v.at[r]],
                                  sem_s).wait()
            return carry

        lax.fori_loop(0, CHUNK_ROWS, drain_body, 0)

    start_in(base_row, ei_a, len_a, sem_a)

    def pair_body(p, carry):
        row_a = base_row + p * (2 * CHUNK_ROWS)
        row_b = row_a + CHUNK_ROWS
        start_in(row_b, ei_b, len_b, sem_b)
        wait_in(row_a, ei_a, len_a, sem_a)
        process(ei_a, len_a)

        @pl.when(p + 1 < n_pair)
        def _():
            start_in(row_b + CHUNK_ROWS, ei_a, len_a, sem_a)

        wait_in(row_b, ei_b, len_b, sem_b)
        process(ei_b, len_b)
        return carry

    lax.fori_loop(0, n_pair, pair_body, 0)

    plsc.subcore_barrier()
    pltpu.sync_copy(acc_sh.at[pl.ds(sid * SLICE, SLICE)], stage_v)
    pltpu.sync_copy(stage_v, out_h.at[pl.ds(cid * ACC_PAD + sid * SLICE, SLICE)])


def _reduce_body(p_h, ae_h, zbl_h, tot_h, a_v, b_v, e_v):
    cid = lax.axis_index("c")
    sid = lax.axis_index("s")
    w = cid * NS + sid
    off = w * P2W
    pltpu.sync_copy(p_h.at[pl.ds(off, P2W)], a_v)
    pltpu.sync_copy(p_h.at[pl.ds(ACC_PAD + off, P2W)], b_v)
    pltpu.sync_copy(ae_h.at[pl.ds(off, P2W)], e_v)

    def vb(v, carry):
        sl = pl.ds(v * L, L)
        z = a_v[sl] + b_v[sl]
        a_v[sl] = z
        e_v[sl] = e_v[sl] + z
        return carry

    lax.fori_loop(0, P2W // L, vb, 0)
    pltpu.sync_copy(a_v, zbl_h.at[pl.ds(off, P2W)])
    pltpu.sync_copy(e_v, tot_h.at[pl.ds(off, P2W)])


_MESH = plsc.VectorSubcoreMesh(core_axis_name="c", subcore_axis_name="s",
                               num_cores=NC, num_subcores=NS)
_CPARAMS = pltpu.CompilerParams(needs_layout_passes=False)

_phase1 = pl.kernel(
    _edge_body,
    out_type=jax.ShapeDtypeStruct((NC * ACC_PAD,), jnp.float32),
    mesh=_MESH,
    scratch_types=[
        pltpu.VMEM((ACC_PAD,), jnp.int32),
        pltpu.VMEM((128,), jnp.float32),
        pltpu.VMEM((128,), jnp.float32),
        pltpu.VMEM((FTAB,), jnp.float32),
        pltpu.VMEM((2, CHUNK_EDGES), jnp.int32),
        pltpu.VMEM((CHUNK_ROWS, ROW_W), jnp.float32),
        pltpu.VMEM((2, CHUNK_EDGES), jnp.int32),
        pltpu.VMEM((CHUNK_ROWS, ROW_W), jnp.float32),
        pltpu.VMEM((CHUNK_ROWS, ROW_W), jnp.int32),
        pltpu.VMEM((CHUNK_ROWS, ROW_W), jnp.float32),
        pltpu.VMEM((SLICE,), jnp.float32),
        pltpu.VMEM_SHARED((ACC_PAD,), jnp.float32),
        pltpu.SemaphoreType.DMA,
        pltpu.SemaphoreType.DMA,
        pltpu.SemaphoreType.DMA,
    ],
    compiler_params=_CPARAMS,
)

_phase2 = pl.kernel(
    _reduce_body,
    out_type=(jax.ShapeDtypeStruct((ACC_PAD,), jnp.float32),
              jax.ShapeDtypeStruct((ACC_PAD,), jnp.float32)),
    mesh=_MESH,
    scratch_types=[
        pltpu.VMEM((P2W,), jnp.float32),
        pltpu.VMEM((P2W,), jnp.float32),
        pltpu.VMEM((P2W,), jnp.float32),
    ],
    compiler_params=_CPARAMS,
)


def kernel(atomic_numbers, edge_index, lengths, atomic_energies,
           c_zbl_raw, d_zbl_raw, zbl_pow_raw, zbl_length_raw):
    an = jnp.zeros((ACC_PAD,), jnp.int32).at[:N_NODES].set(
        atomic_numbers.astype(jnp.int32))
    ei = edge_index.astype(jnp.int32)
    ln = lengths.reshape(ROWS, ROW_W).astype(jnp.float32)

    c = jax.nn.softplus(c_zbl_raw)
    c = c / c.sum()
    d = jax.nn.softplus(d_zbl_raw)
    p = jax.nn.softplus(zbl_pow_raw)[0]
    zl = jax.nn.softplus(zbl_length_raw)[0]

    c = c * jnp.float32(KE / 2.0)
    smax = 2.0 * jnp.float32(94.0) ** p
    tmax = jnp.float32(4.1) * smax / zl
    step = tmax / jnp.float32(FBINS)
    powtab = (jnp.arange(128, dtype=jnp.float32) ** p) / (zl * step)
    covtab = jnp.zeros((128,), jnp.float32).at[:119].set(jnp.asarray(_COV))
    tgrid = jnp.arange(FTAB, dtype=jnp.float32) * step
    ftab = jnp.sum(c[None, :] * jnp.exp(-d[None, :] * tgrid[:, None]), axis=1)

    ae_pad = jnp.zeros((ACC_PAD,), jnp.float32).at[:N_NODES].set(atomic_energies)

    partial = _phase1(an, ei, ln, powtab, covtab, ftab)
    zbl_pad, tot_pad = _phase2(partial, ae_pad)
    return zbl_pad[:N_NODES], tot_pad[:N_NODES]


# speedup vs baseline: 1235.1466x; 1.0001x over previous
import jax
import jax.numpy as jnp
import numpy as np
from jax import lax
from jax.experimental import pallas as pl
from jax.experimental.pallas import tpu as pltpu
from jax.experimental.pallas import tpu_sc as plsc

N_NODES = 100000
N_EDGES = 6400000
KE = 14.399645351950548
_COV = np.linspace(0.2, 2.6, 119).astype(np.float32)

NC, NS, L = 2, 16, 16
NW = NC * NS
ROW_W = 128
ROWS = N_EDGES // ROW_W
CHUNK_ROWS = 8
CHUNK_EDGES = CHUNK_ROWS * 128
ROWS_PER_W = 1568
NPAIR_FULL = ROWS_PER_W // (2 * CHUNK_ROWS)
NPAIR_LAST = (ROWS - (NW - 1) * ROWS_PER_W) // (2 * CHUNK_ROWS)
ACC_PAD = 100352
SLICE = ACC_PAD // NS
P2W = ACC_PAD // NW


FBINS = 4096
FTAB = FBINS + 128


def _edge_body(an_h, ei_h, len_h, pt_h, ct_h, ft_h, out_h,
               an_v, pt_v, ct_v, ft_v,
               ei_a, len_a, ei_b, len_b, ii2_v, val_v,
               stage_v, acc_sh, sem_a, sem_b, sem_s):
    cid = lax.axis_index("c")
    sid = lax.axis_index("s")
    w = cid * NS + sid

    pltpu.sync_copy(an_h, an_v)
    pltpu.sync_copy(pt_h, pt_v)
    pltpu.sync_copy(ct_h, ct_v)
    pltpu.sync_copy(ft_h, ft_v)

    zero = jnp.zeros((L,), jnp.float32)

    def zbody(i, carry):
        stage_v[pl.ds(i * L, L)] = zero
        return carry

    lax.fori_loop(0, SLICE // L, zbody, 0)
    pltpu.sync_copy(stage_v, acc_sh.at[pl.ds(sid * SLICE, SLICE)])
    plsc.subcore_barrier()

    one = jnp.ones((L,), jnp.float32)
    zerov = jnp.zeros((L,), jnp.float32)

    base_row = w * ROWS_PER_W
    n_pair = jnp.where(w == NW - 1, NPAIR_LAST, NPAIR_FULL)

    def start_in(row0, ei_v, len_v, sem):
        pltpu.async_copy(ei_h.at[:, pl.ds(row0 * ROW_W, CHUNK_EDGES)], ei_v, sem)
        pltpu.async_copy(len_h.at[pl.ds(row0, CHUNK_ROWS)], len_v, sem)

    def wait_in(row0, ei_v, len_v, sem):
        pltpu.make_async_copy(ei_h.at[:, pl.ds(row0 * ROW_W, CHUNK_EDGES)], ei_v, sem).wait()
        pltpu.make_async_copy(len_h.at[pl.ds(row0, CHUNK_ROWS)], len_v, sem).wait()

    def process(ei_v, len_v):
        def row_body(r, carry):
            def vbody(v):
                sl = pl.ds(v * L, L)
                esl = pl.ds(r * ROW_W + v * L, L)
                ii = ei_v[0, esl]
                ij = ei_v[1, esl]
                ln = len_v[r, sl]
                zi = plsc.load_gather(an_v, [ii])
                zj = plsc.load_gather(an_v, [ij])
                pi = plsc.load_gather(pt_v, [zi])
                pj = plsc.load_gather(pt_v, [zj])
                ci = plsc.load_gather(ct_v, [zi])
                cj = plsc.load_gather(ct_v, [zj])
                zif = zi.astype(jnp.float32)
                zjf = zj.astype(jnp.float32)
                s = pi + pj
                u = ln * s
                k = u.astype(jnp.int32)
                frac = u - k.astype(jnp.float32)
                f0 = plsc.load_gather(ft_v, [k])
                f1 = plsc.load_gather(ft_v, [k + 1])
                fg = f0 + frac * (f1 - f0)
                pref = zif * zjf * s
                x = ln / (ci + cj)
                x2 = x * x
                x6 = x2 * x2 * x2
                poly = one + x6 * (x * 48.0 - x2 * 21.0 - 28.0)
                val = jnp.where(x < one, pref * fg * poly, zerov)
                val_v[r, sl] = val
                ii2_v[r, sl] = ii

            plsc.parallel_loop(0, ROW_W // L, 1, unroll=8)(vbody)
            pltpu.async_copy(val_v.at[r], acc_sh.at[ii2_v.at[r]], sem_s,
                             add=True)
            return carry

        lax.fori_loop(0, CHUNK_ROWS, row_body, 0)

        def drain_body(r, carry):
            pltpu.make_async_copy(val_v.at[r], acc_sh.at[ii2_v.at[r]],
                                  sem_s).wait()
            return carry

        lax.fori_loop(0, CHUNK_ROWS, drain_body, 0)

    start_in(base_row, ei_a, len_a, sem_a)

    def pair_body(p, carry):
        row_a = base_row + p * (2 * CHUNK_ROWS)
        row_b = row_a + CHUNK_ROWS
        start_in(row_b, ei_b, len_b, sem_b)
        wait_in(row_a, ei_a, len_a, sem_a)
        process(ei_a, len_a)

        @pl.when(p + 1 < n_pair)
        def _():
            start_in(row_b + CHUNK_ROWS, ei_a, len_a, sem_a)

        wait_in(row_b, ei_b, len_b, sem_b)
        process(ei_b, len_b)
        return carry

    lax.fori_loop(0, n_pair, pair_body, 0)

    plsc.subcore_barrier()
    pltpu.sync_copy(acc_sh.at[pl.ds(sid * SLICE, SLICE)], stage_v)
    pltpu.sync_copy(stage_v, out_h.at[pl.ds(cid * ACC_PAD + sid * SLICE, SLICE)])


def _reduce_body(p_h, ae_h, zbl_h, tot_h, a_v, b_v, e_v):
    cid = lax.axis_index("c")
    sid = lax.axis_index("s")
    w = cid * NS + sid
    off = w * P2W
    pltpu.sync_copy(p_h.at[pl.ds(off, P2W)], a_v)
    pltpu.sync_copy(p_h.at[pl.ds(ACC_PAD + off, P2W)], b_v)
    pltpu.sync_copy(ae_h.at[pl.ds(off, P2W)], e_v)

    def vb(v, carry):
        sl = pl.ds(v * L, L)
        z = a_v[sl] + b_v[sl]
        a_v[sl] = z
        e_v[sl] = e_v[sl] + z
        return carry

    lax.fori_loop(0, P2W // L, vb, 0)
    pltpu.sync_copy(a_v, zbl_h.at[pl.ds(off, P2W)])
    pltpu.sync_copy(e_v, tot_h.at[pl.ds(off, P2W)])


_MESH = plsc.VectorSubcoreMesh(core_axis_name="c", subcore_axis_name="s",
                               num_cores=NC, num_subcores=NS)
_CPARAMS = pltpu.CompilerParams(needs_layout_passes=False)

_phase1 = pl.kernel(
    _edge_body,
    out_type=jax.ShapeDtypeStruct((NC * ACC_PAD,), jnp.float32),
    mesh=_MESH,
    scratch_types=[
        pltpu.VMEM((ACC_PAD,), jnp.int32),
        pltpu.VMEM((128,), jnp.float32),
        pltpu.VMEM((128,), jnp.float32),
        pltpu.VMEM((FTAB,), jnp.float32),
        pltpu.VMEM((2, CHUNK_EDGES), jnp.int32),
        pltpu.VMEM((CHUNK_ROWS, ROW_W), jnp.float32),
        pltpu.VMEM((2, CHUNK_EDGES), jnp.int32),
        pltpu.VMEM((CHUNK_ROWS, ROW_W), jnp.float32),
        pltpu.VMEM((CHUNK_ROWS, ROW_W), jnp.int32),
        pltpu.VMEM((CHUNK_ROWS, ROW_W), jnp.float32),
        pltpu.VMEM((SLICE,), jnp.float32),
        pltpu.VMEM_SHARED((ACC_PAD,), jnp.float32),
        pltpu.SemaphoreType.DMA,
        pltpu.SemaphoreType.DMA,
        pltpu.SemaphoreType.DMA,
    ],
    compiler_params=_CPARAMS,
)

_phase2 = pl.kernel(
    _reduce_body,
    out_type=(jax.ShapeDtypeStruct((ACC_PAD,), jnp.float32),
              jax.ShapeDtypeStruct((ACC_PAD,), jnp.float32)),
    mesh=_MESH,
    scratch_types=[
        pltpu.VMEM((P2W,), jnp.float32),
        pltpu.VMEM((P2W,), jnp.float32),
        pltpu.VMEM((P2W,), jnp.float32),
    ],
    compiler_params=_CPARAMS,
)


def kernel(atomic_numbers, edge_index, lengths, atomic_energies,
           c_zbl_raw, d_zbl_raw, zbl_pow_raw, zbl_length_raw):
    an = jnp.zeros((ACC_PAD,), jnp.int32).at[:N_NODES].set(
        atomic_numbers.astype(jnp.int32))
    ei = edge_index.astype(jnp.int32)
    ln = lengths.reshape(ROWS, ROW_W).astype(jnp.float32)

    c = jax.nn.softplus(c_zbl_raw)
    c = c / c.sum()
    d = jax.nn.softplus(d_zbl_raw)
    p = jax.nn.softplus(zbl_pow_raw)[0]
    zl = jax.nn.softplus(zbl_length_raw)[0]

    c = c * jnp.float32(KE / 2.0)
    smax = 2.0 * jnp.float32(94.0) ** p
    tmax = jnp.float32(4.1) * smax / zl
    step = tmax / jnp.float32(FBINS)
    powtab = (jnp.arange(128, dtype=jnp.float32) ** p) / (zl * step)
    covtab = jnp.zeros((128,), jnp.float32).at[:119].set(jnp.asarray(_COV))
    ugrid = jnp.arange(FTAB, dtype=jnp.float32)
    tgrid = ugrid * step
    fvals = jnp.sum(c[None, :] * jnp.exp(-d[None, :] * tgrid[:, None]), axis=1)
    ftab = fvals / jnp.maximum(ugrid, jnp.float32(1.0))

    ae_pad = jnp.zeros((ACC_PAD,), jnp.float32).at[:N_NODES].set(atomic_energies)

    partial = _phase1(an, ei, ln, powtab, covtab, ftab)
    zbl_pad, tot_pad = _phase2(partial, ae_pad)
    return zbl_pad[:N_NODES], tot_pad[:N_NODES]


# speedup vs baseline: 1242.0040x; 1.0056x over previous
import jax
import jax.numpy as jnp
import numpy as np
from jax import lax
from jax.experimental import pallas as pl
from jax.experimental.pallas import tpu as pltpu
from jax.experimental.pallas import tpu_sc as plsc

N_NODES = 100000
N_EDGES = 6400000
KE = 14.399645351950548
_COV = np.linspace(0.2, 2.6, 119).astype(np.float32)
_COV_H = float(np.float32(2.4) / np.float32(118.0))

NC, NS, L = 2, 16, 16
NW = NC * NS
ROW_W = 128
ROWS = N_EDGES // ROW_W
CHUNK_ROWS = 8
CHUNK_EDGES = CHUNK_ROWS * 128
ROWS_PER_W = 1568
NPAIR_FULL = ROWS_PER_W // (2 * CHUNK_ROWS)
NPAIR_LAST = (ROWS - (NW - 1) * ROWS_PER_W) // (2 * CHUNK_ROWS)
ACC_PAD = 100352
SLICE = ACC_PAD // NS
P2W = ACC_PAD // NW


FBINS = 4096
FTAB = FBINS + 128


def _edge_body(an_h, ei_h, len_h, pt_h, ft_h, out_h,
               an_v, pt_v, ft_v,
               ei_a, len_a, ei_b, len_b, ii2_v, val_v,
               stage_v, acc_sh, sem_a, sem_b, sem_s):
    cid = lax.axis_index("c")
    sid = lax.axis_index("s")
    w = cid * NS + sid

    pltpu.sync_copy(an_h, an_v)
    pltpu.sync_copy(pt_h, pt_v)
    pltpu.sync_copy(ft_h, ft_v)

    zero = jnp.zeros((L,), jnp.float32)

    def zbody(i, carry):
        stage_v[pl.ds(i * L, L)] = zero
        return carry

    lax.fori_loop(0, SLICE // L, zbody, 0)
    pltpu.sync_copy(stage_v, acc_sh.at[pl.ds(sid * SLICE, SLICE)])
    plsc.subcore_barrier()

    one = jnp.ones((L,), jnp.float32)
    zerov = jnp.zeros((L,), jnp.float32)

    base_row = w * ROWS_PER_W
    n_pair = jnp.where(w == NW - 1, NPAIR_LAST, NPAIR_FULL)

    def start_in(row0, ei_v, len_v, sem):
        pltpu.async_copy(ei_h.at[:, pl.ds(row0 * ROW_W, CHUNK_EDGES)], ei_v, sem)
        pltpu.async_copy(len_h.at[pl.ds(row0, CHUNK_ROWS)], len_v, sem)

    def wait_in(row0, ei_v, len_v, sem):
        pltpu.make_async_copy(ei_h.at[:, pl.ds(row0 * ROW_W, CHUNK_EDGES)], ei_v, sem).wait()
        pltpu.make_async_copy(len_h.at[pl.ds(row0, CHUNK_ROWS)], len_v, sem).wait()

    def process(ei_v, len_v):
        def row_body(r, carry):
            def vbody(v):
                sl = pl.ds(v * L, L)
                esl = pl.ds(r * ROW_W + v * L, L)
                ii = ei_v[0, esl]
                ij = ei_v[1, esl]
                ln = len_v[r, sl]
                zi = plsc.load_gather(an_v, [ii])
                zj = plsc.load_gather(an_v, [ij])
                pi = plsc.load_gather(pt_v, [zi])
                pj = plsc.load_gather(pt_v, [zj])
                zif = zi.astype(jnp.float32)
                zjf = zj.astype(jnp.float32)
                crad = (zif + zjf) * _COV_H + 0.4
                s = pi + pj
                u = ln * s
                k = u.astype(jnp.int32)
                frac = u - k.astype(jnp.float32)
                f0 = plsc.load_gather(ft_v, [k])
                f1 = plsc.load_gather(ft_v, [k + 1])
                fg = f0 + frac * (f1 - f0)
                pref = zif * zjf * s
                x = ln / crad
                x2 = x * x
                x6 = x2 * x2 * x2
                poly = one + x6 * (x * 48.0 - x2 * 21.0 - 28.0)
                val = jnp.where(x < one, pref * fg * poly, zerov)
                val_v[r, sl] = val
                ii2_v[r, sl] = ii

            plsc.parallel_loop(0, ROW_W // L, 1, unroll=8)(vbody)
            pltpu.async_copy(val_v.at[r], acc_sh.at[ii2_v.at[r]], sem_s,
                             add=True)
            return carry

        lax.fori_loop(0, CHUNK_ROWS, row_body, 0)

        def drain_body(r, carry):
            pltpu.make_async_copy(val_v.at[r], acc_sh.at[ii2_v.at[r]],
                                  sem_s).wait()
            return carry

        lax.fori_loop(0, CHUNK_ROWS, drain_body, 0)

    start_in(base_row, ei_a, len_a, sem_a)

    def pair_body(p, carry):
        row_a = base_row + p * (2 * CHUNK_ROWS)
        row_b = row_a + CHUNK_ROWS
        start_in(row_b, ei_b, len_b, sem_b)
        wait_in(row_a, ei_a, len_a, sem_a)
        process(ei_a, len_a)

        @pl.when(p + 1 < n_pair)
        def _():
            start_in(row_b + CHUNK_ROWS, ei_a, len_a, sem_a)

        wait_in(row_b, ei_b, len_b, sem_b)
        process(ei_b, len_b)
        return carry

    lax.fori_loop(0, n_pair, pair_body, 0)

    plsc.subcore_barrier()
    pltpu.sync_copy(acc_sh.at[pl.ds(sid * SLICE, SLICE)], stage_v)
    pltpu.sync_copy(stage_v, out_h.at[pl.ds(cid * ACC_PAD + sid * SLICE, SLICE)])


def _reduce_body(p_h, ae_h, zbl_h, tot_h, a_v, b_v, e_v):
    cid = lax.axis_index("c")
    sid = lax.axis_index("s")
    w = cid * NS + sid
    off = w * P2W
    pltpu.sync_copy(p_h.at[pl.ds(off, P2W)], a_v)
    pltpu.sync_copy(p_h.at[pl.ds(ACC_PAD + off, P2W)], b_v)
    pltpu.sync_copy(ae_h.at[pl.ds(off, P2W)], e_v)

    def vb(v, carry):
        sl = pl.ds(v * L, L)
        z = a_v[sl] + b_v[sl]
        a_v[sl] = z
        e_v[sl] = e_v[sl] + z
        return carry

    lax.fori_loop(0, P2W // L, vb, 0)
    pltpu.sync_copy(a_v, zbl_h.at[pl.ds(off, P2W)])
    pltpu.sync_copy(e_v, tot_h.at[pl.ds(off, P2W)])


_MESH = plsc.VectorSubcoreMesh(core_axis_name="c", subcore_axis_name="s",
                               num_cores=NC, num_subcores=NS)
_CPARAMS = pltpu.CompilerParams(needs_layout_passes=False)

_phase1 = pl.kernel(
    _edge_body,
    out_type=jax.ShapeDtypeStruct((NC * ACC_PAD,), jnp.float32),
    mesh=_MESH,
    scratch_types=[
        pltpu.VMEM((ACC_PAD,), jnp.int32),
        pltpu.VMEM((128,), jnp.float32),
        pltpu.VMEM((FTAB,), jnp.float32),
        pltpu.VMEM((2, CHUNK_EDGES), jnp.int32),
        pltpu.VMEM((CHUNK_ROWS, ROW_W), jnp.float32),
        pltpu.VMEM((2, CHUNK_EDGES), jnp.int32),
        pltpu.VMEM((CHUNK_ROWS, ROW_W), jnp.float32),
        pltpu.VMEM((CHUNK_ROWS, ROW_W), jnp.int32),
        pltpu.VMEM((CHUNK_ROWS, ROW_W), jnp.float32),
        pltpu.VMEM((SLICE,), jnp.float32),
        pltpu.VMEM_SHARED((ACC_PAD,), jnp.float32),
        pltpu.SemaphoreType.DMA,
        pltpu.SemaphoreType.DMA,
        pltpu.SemaphoreType.DMA,
    ],
    compiler_params=_CPARAMS,
)

_phase2 = pl.kernel(
    _reduce_body,
    out_type=(jax.ShapeDtypeStruct((ACC_PAD,), jnp.float32),
              jax.ShapeDtypeStruct((ACC_PAD,), jnp.float32)),
    mesh=_MESH,
    scratch_types=[
        pltpu.VMEM((P2W,), jnp.float32),
        pltpu.VMEM((P2W,), jnp.float32),
        pltpu.VMEM((P2W,), jnp.float32),
    ],
    compiler_params=_CPARAMS,
)


def kernel(atomic_numbers, edge_index, lengths, atomic_energies,
           c_zbl_raw, d_zbl_raw, zbl_pow_raw, zbl_length_raw):
    an = jnp.zeros((ACC_PAD,), jnp.int32).at[:N_NODES].set(
        atomic_numbers.astype(jnp.int32))
    ei = edge_index.astype(jnp.int32)
    ln = lengths.reshape(ROWS, ROW_W).astype(jnp.float32)

    c = jax.nn.softplus(c_zbl_raw)
    c = c / c.sum()
    d = jax.nn.softplus(d_zbl_raw)
    p = jax.nn.softplus(zbl_pow_raw)[0]
    zl = jax.nn.softplus(zbl_length_raw)[0]

    c = c * jnp.float32(KE / 2.0)
    smax = 2.0 * jnp.float32(94.0) ** p
    tmax = jnp.float32(4.1) * smax / zl
    step = tmax / jnp.float32(FBINS)
    powtab = (jnp.arange(128, dtype=jnp.float32) ** p) / (zl * step)
    ugrid = jnp.arange(FTAB, dtype=jnp.float32)
    tgrid = ugrid * step
    fvals = jnp.sum(c[None, :] * jnp.exp(-d[None, :] * tgrid[:, None]), axis=1)
    ftab = fvals / jnp.maximum(ugrid, jnp.float32(1.0))

    ae_pad = jnp.zeros((ACC_PAD,), jnp.float32).at[:N_NODES].set(atomic_energies)

    partial = _phase1(an, ei, ln, powtab, ftab)
    zbl_pad, tot_pad = _phase2(partial, ae_pad)
    return zbl_pad[:N_NODES], tot_pad[:N_NODES]
